# baseline (speedup 1.0000x reference)
import jax
import jax.numpy as jnp
from jax.experimental import pallas as pl

IN_NF = 128
OUT_NF = 128


def _mlp(params, x):
    n = len(params)
    for i, (W, b) in enumerate(params):
        x = x @ W + b
        if i < n - 1:
            x = jnp.where(x >= 0, x, 0.2 * x)
    return x


def kernel(nf, edge_index_net_out, ef_net_out, edge_index_net_in, ef_net_in, input_nodes, output_nodes, params):
    N = nf.shape[0]
    src_o, dst_o = edge_index_net_out[0], edge_index_net_out[1]
    src_i, dst_i = edge_index_net_in[0], edge_index_net_in[1]
    x = jnp.concatenate([nf[src_o], nf[dst_o], ef_net_out], axis=1)
    x = _mlp(params['msg_out'], x)
    Wf, bf = params['msg_out_fc']
    efi = jnp.concatenate([x, nf[src_o]], axis=1) @ Wf + bf
    nfi = jax.ops.segment_sum(efi, dst_o, num_segments=N)
    red_out_x = _mlp(params['red_out'], jnp.concatenate([nf[input_nodes], nfi[input_nodes]], axis=1))
    new_nf = jnp.zeros((N, OUT_NF), dtype=nf.dtype).at[input_nodes].set(red_out_x)
    x = jnp.concatenate([nf[src_i], nf[dst_i], ef_net_in], axis=1)
    x = _mlp(params['msg_in'], x)
    k = jax.nn.sigmoid(x[:, :1])
    f1 = x[:, 1:1 + IN_NF]
    f2 = x[:, 1 + IN_NF:]
    W1, b1 = params['msg_in_fc1']
    W2, b2 = params['msg_in_fc2']
    x1 = jnp.concatenate([f1 * k, nf[src_i]], axis=1) @ W1 + b1
    x2 = jnp.concatenate([f2 * k, nf[src_i]], axis=1) @ W2 + b2
    cnt = jax.ops.segment_sum(jnp.ones((x1.shape[0],), dtype=nf.dtype), dst_i, num_segments=N)
    nfo1 = jax.ops.segment_sum(x1, dst_i, num_segments=N) / jnp.maximum(cnt, 1.0)[:, None]
    nfo2 = jax.ops.segment_max(x2, dst_i, num_segments=N)
    nfo2 = jnp.where(jnp.isneginf(nfo2), 0.0, nfo2)
    red_in_x = _mlp(params['red_in'], jnp.concatenate([nf[output_nodes], nfo1[output_nodes], nfo2[output_nodes]], axis=1))
    new_nf = new_nf.at[output_nodes].set(red_in_x)
    return new_nf


# speedup vs baseline: 2.5451x; 2.5451x over previous
import functools

import jax
import jax.numpy as jnp
from jax import lax
from jax.experimental import pallas as pl
from jax.experimental.pallas import tpu as pltpu
from jax.experimental.pallas import tpu_sc as plsc

N = 10000
E = 640000
CHUNK = 128
NCH = E // CHUNK
NTILE = 16
NWORK = 32
F32 = jnp.float32
I32 = jnp.int32


def _mesh():
    return plsc.VectorSubcoreMesh(core_axis_name="c", subcore_axis_name="s")


def _k1_body(nf_ref, wpo_ref, wfo_ref, wpi_ref, wf1_ref, wf2_ref,
             po_ref, fo_ref, pi_ref, f1_ref, f2_ref):
    x = nf_ref[...]
    po_ref[...] = jnp.dot(x, wpo_ref[...], preferred_element_type=F32)
    fo_ref[...] = jnp.dot(x, wfo_ref[...], preferred_element_type=F32)
    pi_ref[...] = jnp.dot(x, wpi_ref[...], preferred_element_type=F32)
    f1_ref[...] = jnp.dot(x, wf1_ref[...], preferred_element_type=F32)
    f2_ref[...] = jnp.dot(x, wf2_ref[...], preferred_element_type=F32)


def _node_tables(nf, wpo, wfo, wpi, wf1, wf2):
    blk = 2000
    grid = N // blk
    wspec = pl.BlockSpec((128, 128), lambda i: (0, 0))
    dspec = pl.BlockSpec((blk, 128), lambda i: (i, 0))
    return pl.pallas_call(
        _k1_body,
        grid=(grid,),
        in_specs=[dspec] + [wspec] * 5,
        out_specs=[dspec] * 5,
        out_shape=[jax.ShapeDtypeStruct((N, 128), F32)] * 5,
    )(nf, wpo, wfo, wpi, wf1, wf2)


def _k2_body(po, fo, pi, f1t, f2t, eio, eii,
             gop_s, gop_d, gof, gip_s, gip_d, gif1, gif2,
             idxs, idxd, bs, bd, b2, b3, sem1, sem2, sem3, sem4):
    w = lax.axis_index("s") * 2 + lax.axis_index("c")

    def body(t, carry):
        c = t * NWORK + w

        @pl.when(c < NCH)
        def _():
            base = c * CHUNK
            pltpu.sync_copy(eio.at[0, pl.ds(base, CHUNK)], idxs)
            pltpu.sync_copy(eio.at[1, pl.ds(base, CHUNK)], idxd)
            cp1 = pltpu.async_copy(po.at[idxs], bs, sem1)
            cp2 = pltpu.async_copy(po.at[idxd], bd, sem2)
            cp3 = pltpu.async_copy(fo.at[idxs], b2, sem3)
            cp1.wait()
            cp2.wait()
            cp3.wait()
            pltpu.sync_copy(bs, gop_s.at[pl.ds(base, CHUNK)])
            pltpu.sync_copy(bd, gop_d.at[pl.ds(base, CHUNK)])
            pltpu.sync_copy(b2, gof.at[pl.ds(base, CHUNK)])
            pltpu.sync_copy(eii.at[0, pl.ds(base, CHUNK)], idxs)
            pltpu.sync_copy(eii.at[1, pl.ds(base, CHUNK)], idxd)
            cp4 = pltpu.async_copy(pi.at[idxs], bs, sem1)
            cp5 = pltpu.async_copy(pi.at[idxd], bd, sem2)
            cp6 = pltpu.async_copy(f1t.at[idxs], b2, sem3)
            cp7 = pltpu.async_copy(f2t.at[idxs], b3, sem4)
            cp4.wait()
            cp5.wait()
            cp6.wait()
            cp7.wait()
            pltpu.sync_copy(bs, gip_s.at[pl.ds(base, CHUNK)])
            pltpu.sync_copy(bd, gip_d.at[pl.ds(base, CHUNK)])
            pltpu.sync_copy(b2, gif1.at[pl.ds(base, CHUNK)])
            pltpu.sync_copy(b3, gif2.at[pl.ds(base, CHUNK)])

        return carry

    lax.fori_loop(0, (NCH + NWORK - 1) // NWORK, body, 0)


def _edge_gathers(po, fo, pi, f1t, f2t, eio, eii):
    k = functools.partial(
        pl.kernel,
        out_type=tuple([jax.ShapeDtypeStruct((E, 128), F32)] * 7),
        mesh=_mesh(),
        scratch_types=[
            pltpu.VMEM((CHUNK,), I32),
            pltpu.VMEM((CHUNK,), I32),
            pltpu.VMEM((CHUNK, 128), F32),
            pltpu.VMEM((CHUNK, 128), F32),
            pltpu.VMEM((CHUNK, 128), F32),
            pltpu.VMEM((CHUNK, 128), F32),
            pltpu.SemaphoreType.DMA,
            pltpu.SemaphoreType.DMA,
            pltpu.SemaphoreType.DMA,
            pltpu.SemaphoreType.DMA,
        ],
    )(_k2_body)
    return k(po, fo, pi, f1t, f2t, eio, eii)


def _leaky(x):
    return jnp.where(x >= 0, x, 0.2 * x)


def _k3_body(gop_s, gop_d, gof, ef, wc, b1, w2, b2, w3, b3, w4f, cf, y):
    h = _leaky(gop_s[:, :64] + gop_d[:, 64:128] +
               jnp.dot(ef[...], wc[...], preferred_element_type=F32) + b1[...])
    h = _leaky(jnp.dot(h, w2[...], preferred_element_type=F32) + b2[...])
    h = _leaky(jnp.dot(h, w3[...], preferred_element_type=F32) + b3[...])
    y[...] = (jnp.dot(h, w4f[...], preferred_element_type=F32) + cf[...] +
              gof[...])


def _msg_out(gop_s, gop_d, gof, ef, wc, b1, w2, b2, w3, b3, w4f, cf):
    blk = 5120
    grid = E // blk
    wspec = lambda shp: pl.BlockSpec(shp, lambda i: (0, 0))
    dspec = pl.BlockSpec((blk, 128), lambda i: (i, 0))
    return pl.pallas_call(
        _k3_body,
        grid=(grid,),
        in_specs=[
            dspec, dspec, dspec,
            pl.BlockSpec((blk, 16), lambda i: (i, 0)),
            wspec((16, 64)), wspec((1, 64)),
            wspec((64, 64)), wspec((1, 64)),
            wspec((64, 64)), wspec((1, 64)),
            wspec((64, 128)), wspec((1, 128)),
        ],
        out_specs=dspec,
        out_shape=jax.ShapeDtypeStruct((E, 128), F32),
    )(gop_s, gop_d, gof, ef, wc, b1, w2, b2, w3, b3, w4f, cf)


def _k6_body(gip_s, gip_d, gif1, gif2, ef, wc, b1, w2, b2, w3, b3,
             w4k, b4k, w4f1, b4f1, w4f2, b4f2, w1a, b1p, w2a, b2p,
             y1, x2t):
    h = _leaky(gip_s[:, :64] + gip_d[:, 64:128] +
               jnp.dot(ef[...], wc[...], preferred_element_type=F32) + b1[...])
    h = _leaky(jnp.dot(h, w2[...], preferred_element_type=F32) + b2[...])
    h = _leaky(jnp.dot(h, w3[...], preferred_element_type=F32) + b3[...])
    xk = jnp.dot(h, w4k[...], preferred_element_type=F32) + b4k[...]
    kgate = 1.0 / (1.0 + jnp.exp(-xk))
    f1 = jnp.dot(h, w4f1[...], preferred_element_type=F32) + b4f1[...]
    f2 = jnp.dot(h, w4f2[...], preferred_element_type=F32) + b4f2[...]
    y1[...] = (jnp.dot(f1 * kgate, w1a[...], preferred_element_type=F32) +
               b1p[...] + gif1[...])
    x2 = (jnp.dot(f2 * kgate, w2a[...], preferred_element_type=F32) +
          b2p[...] + gif2[...])
    x2t[...] = x2.T


def _msg_in(gip_s, gip_d, gif1, gif2, ef, weights):
    blk = 5120
    grid = E // blk
    wspec = lambda shp: pl.BlockSpec(shp, lambda i: (0, 0))
    dspec = pl.BlockSpec((blk, 128), lambda i: (i, 0))
    (wc, b1, w2, b2, w3, b3, w4k, b4k, w4f1, b4f1, w4f2, b4f2,
     w1a, b1p, w2a, b2p) = weights
    return pl.pallas_call(
        _k6_body,
        grid=(grid,),
        in_specs=[
            dspec, dspec, dspec, dspec,
            pl.BlockSpec((blk, 16), lambda i: (i, 0)),
            wspec((16, 64)), wspec((1, 64)),
            wspec((64, 64)), wspec((1, 64)),
            wspec((64, 64)), wspec((1, 64)),
            wspec((64, 1)), wspec((1, 1)),
            wspec((64, 128)), wspec((1, 128)),
            wspec((64, 128)), wspec((1, 128)),
            wspec((128, 128)), wspec((1, 128)),
            wspec((128, 128)), wspec((1, 128)),
        ],
        out_specs=[
            dspec,
            pl.BlockSpec((128, blk), lambda i: (0, i)),
        ],
        out_shape=[
            jax.ShapeDtypeStruct((E, 128), F32),
            jax.ShapeDtypeStruct((128, E), F32),
        ],
    )(gip_s, gip_d, gif1, gif2, ef, wc, b1, w2, b2, w3, b3, w4k, b4k,
      w4f1, b4f1, w4f2, b4f2, w1a, b1p, w2a, b2p)


NPAD = 10240
ROWS = 624


def _zero_rows(zbuf, nrows, ncols):
    def zr(r, carry):
        for j in range(ncols // 16):
            zbuf[r, pl.ds(16 * j, 16)] = jnp.zeros((16,), F32)
        return carry
    lax.fori_loop(0, nrows, zr, 0)


def _scatter_sum_body_factory(with_count):
    def body(*refs):
        if with_count:
            (vals, ei, out, cout, acc, cacc, zbuf, vbuf, idxv, onesv, zc) = refs
        else:
            (vals, ei, out, acc, zbuf, vbuf, idxv) = refs
        cid = lax.axis_index("c")
        sid = lax.axis_index("s")
        _zero_rows(zbuf, 16, 128)
        r0 = sid * ROWS

        def zr(kk, carry):
            pltpu.sync_copy(zbuf, acc.at[pl.ds(r0 + 16 * kk, 16), :])
            return carry
        lax.fori_loop(0, ROWS // 16, zr, 0)

        @pl.when(sid == NTILE - 1)
        def _():
            pltpu.sync_copy(zbuf, acc.at[pl.ds(9984, 16), :])

        if with_count:
            def zc_fill(i, carry):
                zc[pl.ds(16 * i, 16)] = jnp.zeros((16,), F32)
                return carry
            lax.fori_loop(0, 40, zc_fill, 0)
            def ones_fill(i, carry):
                onesv[pl.ds(16 * i, 16)] = jnp.ones((16,), F32)
                return carry
            lax.fori_loop(0, 8, ones_fill, 0)
            pltpu.sync_copy(zc, cacc.at[pl.ds(sid * 640, 640)])
        plsc.subcore_barrier()

        half = NCH // 2
        start = cid * half

        def chunk_body(t, carry):
            c = start + t * NTILE + sid

            @pl.when(c < start + half)
            def _():
                base = c * CHUNK
                pltpu.sync_copy(ei.at[1, pl.ds(base, CHUNK)], idxv)
                pltpu.sync_copy(vals.at[pl.ds(base, CHUNK), :], vbuf)
                pltpu.sync_copy(vbuf, acc.at[idxv], add=True)
                if with_count:
                    pltpu.sync_copy(onesv, cacc.at[idxv], add=True)
            return carry

        lax.fori_loop(0, (half + NTILE - 1) // NTILE, chunk_body, 0)
        plsc.subcore_barrier()
        pltpu.sync_copy(acc.at[pl.ds(r0, ROWS), :],
                        out.at[cid, pl.ds(r0, ROWS), :])

        @pl.when(sid == NTILE - 1)
        def _():
            pltpu.sync_copy(acc.at[pl.ds(9984, 16), :],
                            out.at[cid, pl.ds(9984, 16), :])

        if with_count:
            pltpu.sync_copy(cacc.at[pl.ds(sid * 640, 640)],
                            cout.at[cid, pl.ds(sid * 640, 640)])
    return body


def _segment_sum(vals, ei, with_count):
    out_type = [jax.ShapeDtypeStruct((2, N, 128), F32)]
    scratch = [
        pltpu.VMEM_SHARED((N, 128), F32),
        pltpu.VMEM((16, 128), F32),
        pltpu.VMEM((CHUNK, 128), F32),
        pltpu.VMEM((CHUNK,), I32),
    ]
    if with_count:
        out_type.append(jax.ShapeDtypeStruct((2, NPAD), F32))
        scratch = [
            pltpu.VMEM_SHARED((N, 128), F32),
            pltpu.VMEM_SHARED((NPAD,), F32),
            pltpu.VMEM((16, 128), F32),
            pltpu.VMEM((CHUNK, 128), F32),
            pltpu.VMEM((CHUNK,), I32),
            pltpu.VMEM((CHUNK,), F32),
            pltpu.VMEM((640,), F32),
        ]
    k = functools.partial(
        pl.kernel,
        out_type=tuple(out_type),
        mesh=_mesh(),
        scratch_types=scratch,
    )(_scatter_sum_body_factory(with_count))
    return k(vals, ei)


NEG = float("-inf")
K8_CE = 640
K8_HALF = E // 2


def _k8_body(x2t, ei, onodes, g2t, acc, xb, idxb, onb, ob, dupbuf):
    cid = lax.axis_index("c")
    sid = lax.axis_index("s")
    row0 = 8 * sid

    def init(i, carry):
        acc[pl.ds(16 * i, 16)] = jnp.full((16,), NEG, F32)
        return carry
    lax.fori_loop(0, 8 * N // 16, init, 0)

    pltpu.sync_copy(onodes, onb.at[pl.ds(0, 5000)])
    lane = lax.iota(I32, 16)
    tail = onb[pl.ds(4992, 16)]
    onb[pl.ds(4992, 16)] = jnp.where(lane < 8, tail, 0)
    for t in range(7):
        onb[pl.ds(5008 + 16 * t, 16)] = jnp.zeros((16,), I32)

    def chunk(t, carry):
        base = cid * K8_HALF + t * K8_CE
        pltpu.sync_copy(x2t.at[pl.ds(row0, 8), pl.ds(base, K8_CE)], xb)
        pltpu.sync_copy(ei.at[1, pl.ds(base, K8_CE)], idxb)

        def group(g, carry2):
            dst16 = idxb[pl.ds(16 * g, 16)]
            vs = [xb[j, pl.ds(16 * g, 16)] for j in range(8)]
            dstf = [dst16 + j * N for j in range(8)]
            plsc.store_scatter(dupbuf, [dst16], lane)
            win = plsc.load_gather(dupbuf, [dst16]) == lane
            nodup = jnp.sum(jnp.where(win, 1, 0)) == 16

            @pl.when(nodup)
            def _():
                for j in range(8):
                    cur = plsc.load_gather(acc, [dstf[j]])
                    plsc.store_scatter(acc, [dstf[j]],
                                       jnp.maximum(cur, vs[j]))

            @pl.when(jnp.logical_not(nodup))
            def _():
                for k in range(16):
                    m = lane == k
                    for j in range(8):
                        cur = plsc.load_gather(acc, [dstf[j]])
                        plsc.store_scatter(acc, [dstf[j]],
                                           jnp.maximum(cur, vs[j]), mask=m)
            return carry2

        lax.fori_loop(0, K8_CE // 16, group, 0)
        return carry

    lax.fori_loop(0, K8_HALF // K8_CE, chunk, 0)

    for p in range(2):
        def emit(o, carry):
            on16 = onb[pl.ds(2560 * p + 16 * o, 16)]
            for j in range(8):
                g = plsc.load_gather(acc, [on16 + j * N])
                ob[j, pl.ds(16 * o, 16)] = g
            return carry
        lax.fori_loop(0, 160, emit, 0)
        pltpu.sync_copy(ob, g2t.at[cid, pl.ds(row0, 8), pl.ds(2560 * p, 2560)])


def _segment_max_gathered(x2t, ei, onodes):
    k = functools.partial(
        pl.kernel,
        out_type=jax.ShapeDtypeStruct((2, 128, 5120), F32),
        mesh=_mesh(),
        compiler_params=pltpu.CompilerParams(needs_layout_passes=False),
        scratch_types=[
            pltpu.VMEM((8 * N,), F32),
            pltpu.VMEM((8, K8_CE), F32),
            pltpu.VMEM((K8_CE,), I32),
            pltpu.VMEM((5120,), I32),
            pltpu.VMEM((8, 2560), F32),
            pltpu.VMEM((N,), I32),
        ],
    )(_k8_body)
    return k(x2t, ei, onodes)


def _small_gathers(nf, p0, p1, q0, q1, c0, c1, inodes, onodes):
    def body(nf_, p0_, p1_, q0_, q1_, c0_, c1_, in_, on_,
             gin_nf, gin_p0, gin_p1, gout_nf, gout_q0, gout_q1, cg0, cg1,
             idxv, rb, cb, sem):
        w = lax.axis_index("s") * 2 + lax.axis_index("c")

        def do_chunk(c, is_tail):
            base = c * CHUNK
            s = 8 if is_tail else CHUNK

            def load_idx(src_nodes):
                if is_tail:
                    def zf(i, carry):
                        idxv[pl.ds(16 * i, 16)] = jnp.zeros((16,), I32)
                        return carry
                    lax.fori_loop(0, 8, zf, 0)
                    pltpu.sync_copy(src_nodes.at[pl.ds(base, 8)],
                                    idxv.at[pl.ds(0, 8)])
                else:
                    pltpu.sync_copy(src_nodes.at[pl.ds(base, CHUNK)], idxv)

            def gather_rows(tbl, dst):
                pltpu.async_copy(tbl.at[idxv], rb, sem).wait()
                pltpu.sync_copy(rb.at[pl.ds(0, s), :],
                                dst.at[pl.ds(base, s), :])

            def gather_elems(tbl, dst):
                pltpu.async_copy(tbl.at[idxv], cb, sem).wait()
                pltpu.sync_copy(cb.at[pl.ds(0, s)], dst.at[pl.ds(base, s)])

            load_idx(in_)
            gather_rows(nf_, gin_nf)
            gather_rows(p0_, gin_p0)
            gather_rows(p1_, gin_p1)
            load_idx(on_)
            gather_rows(nf_, gout_nf)
            gather_rows(q0_, gout_q0)
            gather_rows(q1_, gout_q1)
            gather_elems(c0_, cg0)
            gather_elems(c1_, cg1)

        for c in range(40):
            wid = c % NWORK
            is_tail = (c == 39)

            @pl.when(w == wid)
            def _(c=c, is_tail=is_tail):
                do_chunk(c, is_tail)

    k = functools.partial(
        pl.kernel,
        out_type=tuple([jax.ShapeDtypeStruct((5000, 128), F32)] * 6 +
                       [jax.ShapeDtypeStruct((5000,), F32)] * 2),
        mesh=_mesh(),
        scratch_types=[
            pltpu.VMEM((CHUNK,), I32),
            pltpu.VMEM((CHUNK, 128), F32),
            pltpu.VMEM((CHUNK,), F32),
            pltpu.SemaphoreType.DMA,
        ],
    )(body)
    return k(nf, p0, p1, q0, q1, c0, c1, inodes, onodes)


def _k9b_body(gin_nf, gin_p0, gin_p1, gout_nf, gout_q0, gout_q1, cg0, cg1,
              g2t0, g2t1, *refs):
    wro = refs[:8]
    wri = refs[8:16]
    rin_ref, rout_ref = refs[16], refs[17]

    def mlp(x, ws):
        (wa, ba, wb, bb, wc, bc, wd, bd) = ws
        h = _leaky(jnp.dot(x, wa[...], preferred_element_type=F32) + ba[...])
        h = _leaky(jnp.dot(h, wb[...], preferred_element_type=F32) + bb[...])
        h = _leaky(jnp.dot(h, wc[...], preferred_element_type=F32) + bc[...])
        return jnp.dot(h, wd[...], preferred_element_type=F32) + bd[...]

    xin = jnp.concatenate([gin_nf[...], gin_p0[...] + gin_p1[...]], axis=1)
    rin_ref[...] = mlp(xin, wro)

    csum = cg0[...] + cg1[...]
    recip = 1.0 / jnp.maximum(csum, 1.0)
    nfo1 = (gout_q0[...] + gout_q1[...]) * recip.T
    g2 = jnp.maximum(g2t0[...], g2t1[...])
    nfo2 = jnp.where(g2 == NEG, 0.0, g2).T
    xout = jnp.concatenate([gout_nf[...], nfo1, nfo2], axis=1)
    rout_ref[...] = mlp(xout, wri)


def _node_mlps(gin_nf, gin_p0, gin_p1, gout_nf, gout_q0, gout_q1,
               cg0, cg1, g2t0, g2t1, wro, wri):
    wspec = lambda a: pl.BlockSpec(a.shape, lambda: (0,) * a.ndim)
    dspec = pl.BlockSpec((5000, 128), lambda: (0, 0))
    cg0 = cg0.reshape(1, 5000)
    cg1 = cg1.reshape(1, 5000)
    args = [gin_nf, gin_p0, gin_p1, gout_nf, gout_q0, gout_q1, cg0, cg1,
            g2t0, g2t1]
    specs = [dspec] * 6 + [pl.BlockSpec((1, 5000), lambda: (0, 0))] * 2 + \
            [pl.BlockSpec((128, 5000), lambda: (0, 0))] * 2
    for wgt in list(wro) + list(wri):
        args.append(wgt)
        specs.append(wspec(wgt))
    return pl.pallas_call(
        _k9b_body,
        in_specs=specs,
        out_specs=[pl.BlockSpec((5000, 128), lambda: (0, 0))] * 2,
        out_shape=[jax.ShapeDtypeStruct((5000, 128), F32)] * 2,
    )(*args)


def _k9c_body(rin, rout, inodes, onodes, new_nf, zbuf, idxv, idx8, rb):
    cid = lax.axis_index("c")
    sid = lax.axis_index("s")

    @pl.when(cid == 0)
    def _():
        _zero_rows(zbuf, 16, 128)
        r0 = sid * ROWS

        def zr(kk, carry):
            pltpu.sync_copy(zbuf, new_nf.at[pl.ds(r0 + 16 * kk, 16), :])
            return carry
        lax.fori_loop(0, ROWS // 16, zr, 0)

        @pl.when(sid == NTILE - 1)
        def _():
            pltpu.sync_copy(zbuf, new_nf.at[pl.ds(9984, 16), :])
        plsc.subcore_barrier()

        def scatter_phase(nodes, rows):
            for c in range(40):
                @pl.when(sid == c % 16)
                def _(c=c):
                    base = c * CHUNK
                    if c == 39:
                        pltpu.sync_copy(nodes.at[pl.ds(base, 8)], idx8)
                        pltpu.sync_copy(rows.at[pl.ds(base, 8), :],
                                        rb.at[pl.ds(0, 8), :])
                        pltpu.sync_copy(rb.at[pl.ds(0, 8), :],
                                        new_nf.at[idx8])
                    else:
                        pltpu.sync_copy(nodes.at[pl.ds(base, CHUNK)], idxv)
                        pltpu.sync_copy(rows.at[pl.ds(base, CHUNK), :], rb)
                        pltpu.sync_copy(rb, new_nf.at[idxv])

        scatter_phase(inodes, rin)
        plsc.subcore_barrier()
        scatter_phase(onodes, rout)


def _assemble(rin, rout, inodes, onodes):
    k = functools.partial(
        pl.kernel,
        out_type=jax.ShapeDtypeStruct((N, 128), F32),
        mesh=_mesh(),
        scratch_types=[
            pltpu.VMEM((16, 128), F32),
            pltpu.VMEM((CHUNK,), I32),
            pltpu.VMEM((8,), I32),
            pltpu.VMEM((CHUNK, 128), F32),
        ],
    )(_k9c_body)
    return k(rin, rout, inodes, onodes)


def kernel(nf, edge_index_net_out, ef_net_out, edge_index_net_in, ef_net_in,
           input_nodes, output_nodes, params):
    (w1o, b1o), (w2o, b2o), (w3o, b3o), (w4o, b4o) = params['msg_out']
    wf, bf = params['msg_out_fc']
    (w1i, b1i), (w2i, b2i), (w3i, b3i), (w4i, b4i) = params['msg_in']
    wfc1, bfc1 = params['msg_in_fc1']
    wfc2, bfc2 = params['msg_in_fc2']

    wpo = jnp.concatenate([w1o[:128], w1o[128:256]], axis=1)
    wfo = wf[128:256]
    wpi = jnp.concatenate([w1i[:128], w1i[128:256]], axis=1)
    wf1t = wfc1[128:256]
    wf2t = wfc2[128:256]
    w4f = jnp.dot(w4o, wf[:128])
    cf = (jnp.dot(b4o, wf[:128]) + bf).reshape(1, 128)
    efw_o = w1o[256:272]
    efw_i = w1i[256:272]
    row = lambda v: v.reshape(1, -1)

    po, fo, pi, f1t, f2t = _node_tables(nf, wpo, wfo, wpi, wf1t, wf2t)

    (gop_s, gop_d, gof, gip_s, gip_d, gif1, gif2) = _edge_gathers(
        po, fo, pi, f1t, f2t, edge_index_net_out, edge_index_net_in)

    y_o = _msg_out(gop_s, gop_d, gof, ef_net_out, efw_o, row(b1o), w2o,
                   row(b2o), w3o, row(b3o), w4f, cf)

    (nfi_p,) = _segment_sum(y_o, edge_index_net_out, with_count=False)

    w4k = w4i[:, 0:1]
    b4k = b4i[0:1].reshape(1, 1)
    w4f1 = w4i[:, 1:129]
    b4f1 = b4i[1:129].reshape(1, 128)
    w4f2 = w4i[:, 129:257]
    b4f2 = b4i[129:257].reshape(1, 128)
    weights_in = (efw_i, row(b1i), w2i, row(b2i), w3i, row(b3i),
                  w4k, b4k, w4f1, b4f1, w4f2, b4f2,
                  wfc1[:128], row(bfc1), wfc2[:128], row(bfc2))
    y1, x2t = _msg_in(gip_s, gip_d, gif1, gif2, ef_net_in, weights_in)

    nfo1_p, cnt_p = _segment_sum(y1, edge_index_net_in, with_count=True)

    g2t = _segment_max_gathered(x2t, edge_index_net_in, output_nodes)
    g2t0 = g2t[0, :, :5000]
    g2t1 = g2t[1, :, :5000]

    p0, p1 = nfi_p[0], nfi_p[1]
    q0, q1 = nfo1_p[0], nfo1_p[1]
    c0, c1 = cnt_p[0], cnt_p[1]
    (gin_nf, gin_p0, gin_p1, gout_nf, gout_q0, gout_q1, cg0, cg1) = \
        _small_gathers(nf, p0, p1, q0, q1, c0, c1, input_nodes, output_nodes)

    (wa, ba), (wb, bb), (wc_, bc), (wd, bd) = params['red_out']
    wro = (wa, row(ba), wb, row(bb), wc_, row(bc), wd, row(bd))
    (xa, ya), (xb, yb), (xc, yc), (xd, yd) = params['red_in']
    wri = (xa, row(ya), xb, row(yb), xc, row(yc), xd, row(yd))
    rin, rout = _node_mlps(gin_nf, gin_p0, gin_p1, gout_nf, gout_q0, gout_q1,
                           cg0, cg1, g2t0, g2t1, wro, wri)

    return _assemble(rin, rout, input_nodes, output_nodes)


# speedup vs baseline: 2.5909x; 1.0180x over previous
import functools

import jax
import jax.numpy as jnp
from jax import lax
from jax.experimental import pallas as pl
from jax.experimental.pallas import tpu as pltpu
from jax.experimental.pallas import tpu_sc as plsc

N = 10000
E = 640000
CHUNK = 128
NCH = E // CHUNK
NTILE = 16
NWORK = 32
F32 = jnp.float32
I32 = jnp.int32


def _mesh():
    return plsc.VectorSubcoreMesh(core_axis_name="c", subcore_axis_name="s")


def _k1_body(nf_ref, wpo_ref, wfo_ref, wpi_ref, wf1_ref, wf2_ref,
             po_ref, fo_ref, pi_ref, f1_ref, f2_ref):
    x = nf_ref[...]
    po_ref[...] = jnp.dot(x, wpo_ref[...], preferred_element_type=F32)
    fo_ref[...] = jnp.dot(x, wfo_ref[...], preferred_element_type=F32)
    pi_ref[...] = jnp.dot(x, wpi_ref[...], preferred_element_type=F32)
    f1_ref[...] = jnp.dot(x, wf1_ref[...], preferred_element_type=F32)
    f2_ref[...] = jnp.dot(x, wf2_ref[...], preferred_element_type=F32)


def _node_tables(nf, wpo, wfo, wpi, wf1, wf2):
    blk = 2000
    grid = N // blk
    wspec = pl.BlockSpec((128, 128), lambda i: (0, 0))
    dspec = pl.BlockSpec((blk, 128), lambda i: (i, 0))
    return pl.pallas_call(
        _k1_body,
        grid=(grid,),
        in_specs=[dspec] + [wspec] * 5,
        out_specs=[dspec] * 5,
        out_shape=[jax.ShapeDtypeStruct((N, 128), F32)] * 5,
    )(nf, wpo, wfo, wpi, wf1, wf2)


def _k2_body(po, pi, f2t, eio, eii,
             gop_s, gop_d, gip_s, gip_d, gif2,
             idxs, idxd, bs, bd, b2, sem1, sem2, sem3):
    w = lax.axis_index("s") * 2 + lax.axis_index("c")

    def body(t, carry):
        c = t * NWORK + w

        @pl.when(c < NCH)
        def _():
            base = c * CHUNK
            pltpu.sync_copy(eio.at[0, pl.ds(base, CHUNK)], idxs)
            pltpu.sync_copy(eio.at[1, pl.ds(base, CHUNK)], idxd)
            cp1 = pltpu.async_copy(po.at[idxs], bs, sem1)
            cp2 = pltpu.async_copy(po.at[idxd], bd, sem2)
            cp1.wait()
            cp2.wait()
            pltpu.sync_copy(bs, gop_s.at[pl.ds(base, CHUNK)])
            pltpu.sync_copy(bd, gop_d.at[pl.ds(base, CHUNK)])
            pltpu.sync_copy(eii.at[0, pl.ds(base, CHUNK)], idxs)
            pltpu.sync_copy(eii.at[1, pl.ds(base, CHUNK)], idxd)
            cp4 = pltpu.async_copy(pi.at[idxs], bs, sem1)
            cp5 = pltpu.async_copy(pi.at[idxd], bd, sem2)
            cp6 = pltpu.async_copy(f2t.at[idxs], b2, sem3)
            cp4.wait()
            cp5.wait()
            cp6.wait()
            pltpu.sync_copy(bs, gip_s.at[pl.ds(base, CHUNK)])
            pltpu.sync_copy(bd, gip_d.at[pl.ds(base, CHUNK)])
            pltpu.sync_copy(b2, gif2.at[pl.ds(base, CHUNK)])

        return carry

    lax.fori_loop(0, (NCH + NWORK - 1) // NWORK, body, 0)


def _edge_gathers(po, pi, f2t, eio, eii):
    k = functools.partial(
        pl.kernel,
        out_type=tuple([jax.ShapeDtypeStruct((E, 128), F32)] * 5),
        mesh=_mesh(),
        scratch_types=[
            pltpu.VMEM((CHUNK,), I32),
            pltpu.VMEM((CHUNK,), I32),
            pltpu.VMEM((CHUNK, 128), F32),
            pltpu.VMEM((CHUNK, 128), F32),
            pltpu.VMEM((CHUNK, 128), F32),
            pltpu.SemaphoreType.DMA,
            pltpu.SemaphoreType.DMA,
            pltpu.SemaphoreType.DMA,
        ],
    )(_k2_body)
    return k(po, pi, f2t, eio, eii)


def _leaky(x):
    return jnp.where(x >= 0, x, 0.2 * x)


def _k3_body(gop_s, gop_d, ef, wc, b1, w2, b2, w3, b3, w4f, cf, y):
    h = _leaky(gop_s[:, :64] + gop_d[:, 64:128] +
               jnp.dot(ef[...], wc[...], preferred_element_type=F32) + b1[...])
    h = _leaky(jnp.dot(h, w2[...], preferred_element_type=F32) + b2[...])
    h = _leaky(jnp.dot(h, w3[...], preferred_element_type=F32) + b3[...])
    y[...] = jnp.dot(h, w4f[...], preferred_element_type=F32) + cf[...]


def _msg_out(gop_s, gop_d, ef, wc, b1, w2, b2, w3, b3, w4f, cf):
    blk = 5120
    grid = E // blk
    wspec = lambda shp: pl.BlockSpec(shp, lambda i: (0, 0))
    dspec = pl.BlockSpec((blk, 128), lambda i: (i, 0))
    return pl.pallas_call(
        _k3_body,
        grid=(grid,),
        in_specs=[
            dspec, dspec,
            pl.BlockSpec((blk, 16), lambda i: (i, 0)),
            wspec((16, 64)), wspec((1, 64)),
            wspec((64, 64)), wspec((1, 64)),
            wspec((64, 64)), wspec((1, 64)),
            wspec((64, 128)), wspec((1, 128)),
        ],
        out_specs=dspec,
        out_shape=jax.ShapeDtypeStruct((E, 128), F32),
    )(gop_s, gop_d, ef, wc, b1, w2, b2, w3, b3, w4f, cf)


def _k6_body(gip_s, gip_d, gif2, ef, wc, b1, w2, b2, w3, b3,
             w4k, b4k, w4f1, b4f1, w4f2, b4f2, w1a, b1p, w2a, b2p,
             y1, x2t):
    h = _leaky(gip_s[:, :64] + gip_d[:, 64:128] +
               jnp.dot(ef[...], wc[...], preferred_element_type=F32) + b1[...])
    h = _leaky(jnp.dot(h, w2[...], preferred_element_type=F32) + b2[...])
    h = _leaky(jnp.dot(h, w3[...], preferred_element_type=F32) + b3[...])
    xk = jnp.dot(h, w4k[...], preferred_element_type=F32) + b4k[...]
    kgate = 1.0 / (1.0 + jnp.exp(-xk))
    f1 = jnp.dot(h, w4f1[...], preferred_element_type=F32) + b4f1[...]
    f2 = jnp.dot(h, w4f2[...], preferred_element_type=F32) + b4f2[...]
    y1[...] = (jnp.dot(f1 * kgate, w1a[...], preferred_element_type=F32) +
               b1p[...])
    x2 = (jnp.dot(f2 * kgate, w2a[...], preferred_element_type=F32) +
          b2p[...] + gif2[...])
    x2t[...] = x2.T


def _msg_in(gip_s, gip_d, gif2, ef, weights):
    blk = 5120
    grid = E // blk
    wspec = lambda shp: pl.BlockSpec(shp, lambda i: (0, 0))
    dspec = pl.BlockSpec((blk, 128), lambda i: (i, 0))
    (wc, b1, w2, b2, w3, b3, w4k, b4k, w4f1, b4f1, w4f2, b4f2,
     w1a, b1p, w2a, b2p) = weights
    return pl.pallas_call(
        _k6_body,
        grid=(grid,),
        in_specs=[
            dspec, dspec, dspec,
            pl.BlockSpec((blk, 16), lambda i: (i, 0)),
            wspec((16, 64)), wspec((1, 64)),
            wspec((64, 64)), wspec((1, 64)),
            wspec((64, 64)), wspec((1, 64)),
            wspec((64, 1)), wspec((1, 1)),
            wspec((64, 128)), wspec((1, 128)),
            wspec((64, 128)), wspec((1, 128)),
            wspec((128, 128)), wspec((1, 128)),
            wspec((128, 128)), wspec((1, 128)),
        ],
        out_specs=[
            dspec,
            pl.BlockSpec((128, blk), lambda i: (0, i)),
        ],
        out_shape=[
            jax.ShapeDtypeStruct((E, 128), F32),
            jax.ShapeDtypeStruct((128, E), F32),
        ],
    )(gip_s, gip_d, gif2, ef, wc, b1, w2, b2, w3, b3, w4k, b4k,
      w4f1, b4f1, w4f2, b4f2, w1a, b1p, w2a, b2p)


NPAD = 10240
ROWS = 624


def _zero_rows(zbuf, nrows, ncols):
    def zr(r, carry):
        for j in range(ncols // 16):
            zbuf[r, pl.ds(16 * j, 16)] = jnp.zeros((16,), F32)
        return carry
    lax.fori_loop(0, nrows, zr, 0)


def _scatter_sum_body_factory(with_count):
    def body(*refs):
        if with_count:
            (vals, tbl, ei, out, cout, acc, cacc, zbuf, vbuf, idxv,
             onesv, zc, idxs, gbuf, sem) = refs
        else:
            (vals, tbl, ei, out, acc, zbuf, vbuf, idxv, idxs, gbuf, sem) = refs
        cid = lax.axis_index("c")
        sid = lax.axis_index("s")
        _zero_rows(zbuf, 16, 128)
        r0 = sid * ROWS

        def zr(kk, carry):
            pltpu.sync_copy(zbuf, acc.at[pl.ds(r0 + 16 * kk, 16), :])
            return carry
        lax.fori_loop(0, ROWS // 16, zr, 0)

        @pl.when(sid == NTILE - 1)
        def _():
            pltpu.sync_copy(zbuf, acc.at[pl.ds(9984, 16), :])

        if with_count:
            def zc_fill(i, carry):
                zc[pl.ds(16 * i, 16)] = jnp.zeros((16,), F32)
                return carry
            lax.fori_loop(0, 40, zc_fill, 0)
            def ones_fill(i, carry):
                onesv[pl.ds(16 * i, 16)] = jnp.ones((16,), F32)
                return carry
            lax.fori_loop(0, 8, ones_fill, 0)
            pltpu.sync_copy(zc, cacc.at[pl.ds(sid * 640, 640)])
        plsc.subcore_barrier()

        half = NCH // 2
        start = cid * half

        def chunk_body(t, carry):
            c = start + t * NTILE + sid

            @pl.when(c < start + half)
            def _():
                base = c * CHUNK
                pltpu.sync_copy(ei.at[0, pl.ds(base, CHUNK)], idxs)
                pltpu.sync_copy(ei.at[1, pl.ds(base, CHUNK)], idxv)
                cp = pltpu.async_copy(tbl.at[idxs], gbuf, sem)
                pltpu.sync_copy(vals.at[pl.ds(base, CHUNK), :], vbuf)
                pltpu.sync_copy(vbuf, acc.at[idxv], add=True)
                cp.wait()
                pltpu.sync_copy(gbuf, acc.at[idxv], add=True)
                if with_count:
                    pltpu.sync_copy(onesv, cacc.at[idxv], add=True)
            return carry

        lax.fori_loop(0, (half + NTILE - 1) // NTILE, chunk_body, 0)
        plsc.subcore_barrier()
        pltpu.sync_copy(acc.at[pl.ds(r0, ROWS), :],
                        out.at[cid, pl.ds(r0, ROWS), :])

        @pl.when(sid == NTILE - 1)
        def _():
            pltpu.sync_copy(acc.at[pl.ds(9984, 16), :],
                            out.at[cid, pl.ds(9984, 16), :])

        if with_count:
            pltpu.sync_copy(cacc.at[pl.ds(sid * 640, 640)],
                            cout.at[cid, pl.ds(sid * 640, 640)])
    return body


def _segment_sum(vals, tbl, ei, with_count):
    out_type = [jax.ShapeDtypeStruct((2, N, 128), F32)]
    gather_scratch = [
        pltpu.VMEM((CHUNK,), I32),
        pltpu.VMEM((CHUNK, 128), F32),
        pltpu.SemaphoreType.DMA,
    ]
    scratch = [
        pltpu.VMEM_SHARED((N, 128), F32),
        pltpu.VMEM((16, 128), F32),
        pltpu.VMEM((CHUNK, 128), F32),
        pltpu.VMEM((CHUNK,), I32),
    ] + gather_scratch
    if with_count:
        out_type.append(jax.ShapeDtypeStruct((2, NPAD), F32))
        scratch = [
            pltpu.VMEM_SHARED((N, 128), F32),
            pltpu.VMEM_SHARED((NPAD,), F32),
            pltpu.VMEM((16, 128), F32),
            pltpu.VMEM((CHUNK, 128), F32),
            pltpu.VMEM((CHUNK,), I32),
            pltpu.VMEM((CHUNK,), F32),
            pltpu.VMEM((640,), F32),
        ] + gather_scratch
    k = functools.partial(
        pl.kernel,
        out_type=tuple(out_type),
        mesh=_mesh(),
        scratch_types=scratch,
    )(_scatter_sum_body_factory(with_count))
    return k(vals, tbl, ei)


NEG = float("-inf")
K8_CE = 640
K8_HALF = E // 2


def _k8_body(x2t, ei, onodes, g2t, acc, xb, idxb, onb, ob, dupbuf):
    cid = lax.axis_index("c")
    sid = lax.axis_index("s")
    row0 = 8 * sid

    def init(i, carry):
        acc[pl.ds(16 * i, 16)] = jnp.full((16,), NEG, F32)
        return carry
    lax.fori_loop(0, 8 * N // 16, init, 0)

    pltpu.sync_copy(onodes, onb.at[pl.ds(0, 5000)])
    lane = lax.iota(I32, 16)
    tail = onb[pl.ds(4992, 16)]
    onb[pl.ds(4992, 16)] = jnp.where(lane < 8, tail, 0)
    for t in range(7):
        onb[pl.ds(5008 + 16 * t, 16)] = jnp.zeros((16,), I32)

    def chunk(t, carry):
        base = cid * K8_HALF + t * K8_CE
        pltpu.sync_copy(x2t.at[pl.ds(row0, 8), pl.ds(base, K8_CE)], xb)
        pltpu.sync_copy(ei.at[1, pl.ds(base, K8_CE)], idxb)

        def group(g, carry2):
            dst16 = idxb[pl.ds(16 * g, 16)]
            vs = [xb[j, pl.ds(16 * g, 16)] for j in range(8)]
            dstf = [dst16 + j * N for j in range(8)]
            plsc.store_scatter(dupbuf, [dst16], lane)
            win = plsc.load_gather(dupbuf, [dst16]) == lane
            nodup = jnp.sum(jnp.where(win, 1, 0)) == 16

            @pl.when(nodup)
            def _():
                for j in range(8):
                    cur = plsc.load_gather(acc, [dstf[j]])
                    plsc.store_scatter(acc, [dstf[j]],
                                       jnp.maximum(cur, vs[j]))

            @pl.when(jnp.logical_not(nodup))
            def _():
                for k in range(16):
                    m = lane == k
                    for j in range(8):
                        cur = plsc.load_gather(acc, [dstf[j]])
                        plsc.store_scatter(acc, [dstf[j]],
                                           jnp.maximum(cur, vs[j]), mask=m)
            return carry2

        lax.fori_loop(0, K8_CE // 16, group, 0)
        return carry

    lax.fori_loop(0, K8_HALF // K8_CE, chunk, 0)

    for p in range(2):
        def emit(o, carry):
            on16 = onb[pl.ds(2560 * p + 16 * o, 16)]
            for j in range(8):
                g = plsc.load_gather(acc, [on16 + j * N])
                ob[j, pl.ds(16 * o, 16)] = g
            return carry
        lax.fori_loop(0, 160, emit, 0)
        pltpu.sync_copy(ob, g2t.at[cid, pl.ds(row0, 8), pl.ds(2560 * p, 2560)])


def _segment_max_gathered(x2t, ei, onodes):
    k = functools.partial(
        pl.kernel,
        out_type=jax.ShapeDtypeStruct((2, 128, 5120), F32),
        mesh=_mesh(),
        compiler_params=pltpu.CompilerParams(needs_layout_passes=False),
        scratch_types=[
            pltpu.VMEM((8 * N,), F32),
            pltpu.VMEM((8, K8_CE), F32),
            pltpu.VMEM((K8_CE,), I32),
            pltpu.VMEM((5120,), I32),
            pltpu.VMEM((8, 2560), F32),
            pltpu.VMEM((N,), I32),
        ],
    )(_k8_body)
    return k(x2t, ei, onodes)


def _small_gathers(nf, p0, p1, q0, q1, c0, c1, inodes, onodes):
    def body(nf_, p0_, p1_, q0_, q1_, c0_, c1_, in_, on_,
             gin_nf, gin_p0, gin_p1, gout_nf, gout_q0, gout_q1, cg0, cg1,
             idxv, rb, cb, sem):
        w = lax.axis_index("s") * 2 + lax.axis_index("c")

        def do_chunk(c, is_tail):
            base = c * CHUNK
            s = 8 if is_tail else CHUNK

            def load_idx(src_nodes):
                if is_tail:
                    def zf(i, carry):
                        idxv[pl.ds(16 * i, 16)] = jnp.zeros((16,), I32)
                        return carry
                    lax.fori_loop(0, 8, zf, 0)
                    pltpu.sync_copy(src_nodes.at[pl.ds(base, 8)],
                                    idxv.at[pl.ds(0, 8)])
                else:
                    pltpu.sync_copy(src_nodes.at[pl.ds(base, CHUNK)], idxv)

            def gather_rows(tbl, dst):
                pltpu.async_copy(tbl.at[idxv], rb, sem).wait()
                pltpu.sync_copy(rb.at[pl.ds(0, s), :],
                                dst.at[pl.ds(base, s), :])

            def gather_elems(tbl, dst):
                pltpu.async_copy(tbl.at[idxv], cb, sem).wait()
                pltpu.sync_copy(cb.at[pl.ds(0, s)], dst.at[pl.ds(base, s)])

            load_idx(in_)
            gather_rows(nf_, gin_nf)
            gather_rows(p0_, gin_p0)
            gather_rows(p1_, gin_p1)
            load_idx(on_)
            gather_rows(nf_, gout_nf)
            gather_rows(q0_, gout_q0)
            gather_rows(q1_, gout_q1)
            gather_elems(c0_, cg0)
            gather_elems(c1_, cg1)

        for c in range(40):
            wid = c % NWORK
            is_tail = (c == 39)

            @pl.when(w == wid)
            def _(c=c, is_tail=is_tail):
                do_chunk(c, is_tail)

    k = functools.partial(
        pl.kernel,
        out_type=tuple([jax.ShapeDtypeStruct((5000, 128), F32)] * 6 +
                       [jax.ShapeDtypeStruct((5000,), F32)] * 2),
        mesh=_mesh(),
        scratch_types=[
            pltpu.VMEM((CHUNK,), I32),
            pltpu.VMEM((CHUNK, 128), F32),
            pltpu.VMEM((CHUNK,), F32),
            pltpu.SemaphoreType.DMA,
        ],
    )(body)
    return k(nf, p0, p1, q0, q1, c0, c1, inodes, onodes)


def _k9b_body(gin_nf, gin_p0, gin_p1, gout_nf, gout_q0, gout_q1, cg0, cg1,
              g2t0, g2t1, *refs):
    wro = refs[:8]
    wri = refs[8:16]
    rin_ref, rout_ref = refs[16], refs[17]

    def mlp(x, ws):
        (wa, ba, wb, bb, wc, bc, wd, bd) = ws
        h = _leaky(jnp.dot(x, wa[...], preferred_element_type=F32) + ba[...])
        h = _leaky(jnp.dot(h, wb[...], preferred_element_type=F32) + bb[...])
        h = _leaky(jnp.dot(h, wc[...], preferred_element_type=F32) + bc[...])
        return jnp.dot(h, wd[...], preferred_element_type=F32) + bd[...]

    xin = jnp.concatenate([gin_nf[...], gin_p0[...] + gin_p1[...]], axis=1)
    rin_ref[...] = mlp(xin, wro)

    csum = cg0[...] + cg1[...]
    recip = 1.0 / jnp.maximum(csum, 1.0)
    nfo1 = (gout_q0[...] + gout_q1[...]) * recip.T
    g2 = jnp.maximum(g2t0[...], g2t1[...])
    nfo2 = jnp.where(g2 == NEG, 0.0, g2).T
    xout = jnp.concatenate([gout_nf[...], nfo1, nfo2], axis=1)
    rout_ref[...] = mlp(xout, wri)


def _node_mlps(gin_nf, gin_p0, gin_p1, gout_nf, gout_q0, gout_q1,
               cg0, cg1, g2t0, g2t1, wro, wri):
    wspec = lambda a: pl.BlockSpec(a.shape, lambda: (0,) * a.ndim)
    dspec = pl.BlockSpec((5000, 128), lambda: (0, 0))
    cg0 = cg0.reshape(1, 5000)
    cg1 = cg1.reshape(1, 5000)
    args = [gin_nf, gin_p0, gin_p1, gout_nf, gout_q0, gout_q1, cg0, cg1,
            g2t0, g2t1]
    specs = [dspec] * 6 + [pl.BlockSpec((1, 5000), lambda: (0, 0))] * 2 + \
            [pl.BlockSpec((128, 5000), lambda: (0, 0))] * 2
    for wgt in list(wro) + list(wri):
        args.append(wgt)
        specs.append(wspec(wgt))
    return pl.pallas_call(
        _k9b_body,
        in_specs=specs,
        out_specs=[pl.BlockSpec((5000, 128), lambda: (0, 0))] * 2,
        out_shape=[jax.ShapeDtypeStruct((5000, 128), F32)] * 2,
    )(*args)


def _k9c_body(rin, rout, inodes, onodes, new_nf, zbuf, idxv, idx8, rb):
    cid = lax.axis_index("c")
    sid = lax.axis_index("s")

    @pl.when(cid == 0)
    def _():
        _zero_rows(zbuf, 16, 128)
        r0 = sid * ROWS

        def zr(kk, carry):
            pltpu.sync_copy(zbuf, new_nf.at[pl.ds(r0 + 16 * kk, 16), :])
            return carry
        lax.fori_loop(0, ROWS // 16, zr, 0)

        @pl.when(sid == NTILE - 1)
        def _():
            pltpu.sync_copy(zbuf, new_nf.at[pl.ds(9984, 16), :])
        plsc.subcore_barrier()

        def scatter_phase(nodes, rows):
            for c in range(40):
                @pl.when(sid == c % 16)
                def _(c=c):
                    base = c * CHUNK
                    if c == 39:
                        pltpu.sync_copy(nodes.at[pl.ds(base, 8)], idx8)
                        pltpu.sync_copy(rows.at[pl.ds(base, 8), :],
                                        rb.at[pl.ds(0, 8), :])
                        pltpu.sync_copy(rb.at[pl.ds(0, 8), :],
                                        new_nf.at[idx8])
                    else:
                        pltpu.sync_copy(nodes.at[pl.ds(base, CHUNK)], idxv)
                        pltpu.sync_copy(rows.at[pl.ds(base, CHUNK), :], rb)
                        pltpu.sync_copy(rb, new_nf.at[idxv])

        scatter_phase(inodes, rin)
        plsc.subcore_barrier()
        scatter_phase(onodes, rout)


def _assemble(rin, rout, inodes, onodes):
    k = functools.partial(
        pl.kernel,
        out_type=jax.ShapeDtypeStruct((N, 128), F32),
        mesh=_mesh(),
        scratch_types=[
            pltpu.VMEM((16, 128), F32),
            pltpu.VMEM((CHUNK,), I32),
            pltpu.VMEM((8,), I32),
            pltpu.VMEM((CHUNK, 128), F32),
        ],
    )(_k9c_body)
    return k(rin, rout, inodes, onodes)


def kernel(nf, edge_index_net_out, ef_net_out, edge_index_net_in, ef_net_in,
           input_nodes, output_nodes, params):
    (w1o, b1o), (w2o, b2o), (w3o, b3o), (w4o, b4o) = params['msg_out']
    wf, bf = params['msg_out_fc']
    (w1i, b1i), (w2i, b2i), (w3i, b3i), (w4i, b4i) = params['msg_in']
    wfc1, bfc1 = params['msg_in_fc1']
    wfc2, bfc2 = params['msg_in_fc2']

    wpo = jnp.concatenate([w1o[:128], w1o[128:256]], axis=1)
    wfo = wf[128:256]
    wpi = jnp.concatenate([w1i[:128], w1i[128:256]], axis=1)
    wf1t = wfc1[128:256]
    wf2t = wfc2[128:256]
    w4f = jnp.dot(w4o, wf[:128])
    cf = (jnp.dot(b4o, wf[:128]) + bf).reshape(1, 128)
    efw_o = w1o[256:272]
    efw_i = w1i[256:272]
    row = lambda v: v.reshape(1, -1)

    po, fo, pi, f1t, f2t = _node_tables(nf, wpo, wfo, wpi, wf1t, wf2t)

    (gop_s, gop_d, gip_s, gip_d, gif2) = _edge_gathers(
        po, pi, f2t, edge_index_net_out, edge_index_net_in)

    y_o = _msg_out(gop_s, gop_d, ef_net_out, efw_o, row(b1o), w2o,
                   row(b2o), w3o, row(b3o), w4f, cf)

    (nfi_p,) = _segment_sum(y_o, fo, edge_index_net_out, with_count=False)

    w4k = w4i[:, 0:1]
    b4k = b4i[0:1].reshape(1, 1)
    w4f1 = w4i[:, 1:129]
    b4f1 = b4i[1:129].reshape(1, 128)
    w4f2 = w4i[:, 129:257]
    b4f2 = b4i[129:257].reshape(1, 128)
    weights_in = (efw_i, row(b1i), w2i, row(b2i), w3i, row(b3i),
                  w4k, b4k, w4f1, b4f1, w4f2, b4f2,
                  wfc1[:128], row(bfc1), wfc2[:128], row(bfc2))
    y1, x2t = _msg_in(gip_s, gip_d, gif2, ef_net_in, weights_in)

    nfo1_p, cnt_p = _segment_sum(y1, f1t, edge_index_net_in, with_count=True)

    g2t = _segment_max_gathered(x2t, edge_index_net_in, output_nodes)
    g2t0 = g2t[0, :, :5000]
    g2t1 = g2t[1, :, :5000]

    p0, p1 = nfi_p[0], nfi_p[1]
    q0, q1 = nfo1_p[0], nfo1_p[1]
    c0, c1 = cnt_p[0], cnt_p[1]
    (gin_nf, gin_p0, gin_p1, gout_nf, gout_q0, gout_q1, cg0, cg1) = \
        _small_gathers(nf, p0, p1, q0, q1, c0, c1, input_nodes, output_nodes)

    (wa, ba), (wb, bb), (wc_, bc), (wd, bd) = params['red_out']
    wro = (wa, row(ba), wb, row(bb), wc_, row(bc), wd, row(bd))
    (xa, ya), (xb, yb), (xc, yc), (xd, yd) = params['red_in']
    wri = (xa, row(ya), xb, row(yb), xc, row(yc), xd, row(yd))
    rin, rout = _node_mlps(gin_nf, gin_p0, gin_p1, gout_nf, gout_q0, gout_q1,
                           cg0, cg1, g2t0, g2t1, wro, wri)

    return _assemble(rin, rout, input_nodes, output_nodes)
